# baseline (speedup 1.0000x reference)
import jax
import jax.numpy as jnp
from jax import lax
from jax.experimental import pallas as pl
from jax.experimental.pallas import tpu as pltpu
from jax.experimental.pallas import tpu_sc as plsc

D_MODEL = 768
NUM_EXPERTS = 64
N_TOKENS = 128
E_BLK = 2
N_STEPS = NUM_EXPERTS // E_BLK
LANES = 16
N_WORKERS = N_TOKENS // LANES


def _logits_kernel(x_ref, gate_ref, out_ref):
    out_ref[...] = jax.lax.dot_general(
        gate_ref[...], x_ref[...], (((1,), (1,)), ((), ())),
        preferred_element_type=jnp.float32)


def _route_sc_kernel(lg_hbm, i1_hbm, i2_hbm, p1_hbm, p2_hbm,
                     lg_v, i1_v, i2_v, p1_v, p2_v, sem):
    wid = lax.axis_index("s") * 2 + lax.axis_index("c")

    @pl.when(wid < N_WORKERS)
    def _():
        pltpu.async_copy(lg_hbm, lg_v, sem).wait()
        off = wid * LANES
        neg = jnp.full((LANES,), -jnp.inf, jnp.float32)
        m1 = neg
        i1 = jnp.zeros((LANES,), jnp.int32)
        for e in range(NUM_EXPERTS):
            v = lg_v[pl.ds(e * N_TOKENS + off, LANES)]
            gt = v > m1
            m1 = jnp.where(gt, v, m1)
            i1 = jnp.where(gt, jnp.full((LANES,), e, jnp.int32), i1)
        m2 = neg
        i2 = jnp.zeros((LANES,), jnp.int32)
        for e in range(NUM_EXPERTS):
            v = lg_v[pl.ds(e * N_TOKENS + off, LANES)]
            v = jnp.where(i1 == e, neg, v)
            gt = v > m2
            m2 = jnp.where(gt, v, m2)
            i2 = jnp.where(gt, jnp.full((LANES,), e, jnp.int32), i2)
        t = jnp.exp(m2 - m1)
        i1_v[...] = i1
        i2_v[...] = i2
        p1_v[...] = 1.0 / (1.0 + t)
        p2_v[...] = t / (1.0 + t)
        pltpu.sync_copy(i1_v, i1_hbm.at[pl.ds(off, LANES)])
        pltpu.sync_copy(i2_v, i2_hbm.at[pl.ds(off, LANES)])
        pltpu.sync_copy(p1_v, p1_hbm.at[pl.ds(off, LANES)])
        pltpu.sync_copy(p2_v, p2_hbm.at[pl.ds(off, LANES)])


def _route(logits_t):
    kern = pl.kernel(
        _route_sc_kernel,
        out_type=(jax.ShapeDtypeStruct((N_TOKENS,), jnp.int32),
                  jax.ShapeDtypeStruct((N_TOKENS,), jnp.int32),
                  jax.ShapeDtypeStruct((N_TOKENS,), jnp.float32),
                  jax.ShapeDtypeStruct((N_TOKENS,), jnp.float32)),
        mesh=plsc.VectorSubcoreMesh(core_axis_name="c", subcore_axis_name="s"),
        scratch_types=[pltpu.VMEM((NUM_EXPERTS * N_TOKENS,), jnp.float32),
                       pltpu.VMEM((LANES,), jnp.int32),
                       pltpu.VMEM((LANES,), jnp.int32),
                       pltpu.VMEM((LANES,), jnp.float32),
                       pltpu.VMEM((LANES,), jnp.float32),
                       pltpu.SemaphoreType.DMA],
    )
    return kern(logits_t.reshape(NUM_EXPERTS * N_TOKENS))


def _moe_kernel(x_ref, w1_ref, b1_ref, w2_ref, b2_ref,
                i1_ref, i2_ref, p1_ref, p2_ref, out_ref, acc_ref):
    s = pl.program_id(0)

    @pl.when(s == 0)
    def _init():
        acc_ref[...] = jnp.zeros_like(acc_ref)

    xb = x_ref[...].astype(jnp.bfloat16)
    contrib = acc_ref[...]
    for k in range(E_BLK):
        e = s * E_BLK + k
        w1 = w1_ref[k].astype(jnp.bfloat16)
        h = jax.lax.dot_general(xb, w1, (((1,), (0,)), ((), ())),
                                preferred_element_type=jnp.float32)
        h += b1_ref[k, 0]
        h = h * 0.5 * (1.0 + jax.lax.erf(h * 0.7071067811865476))
        w2 = w2_ref[k].astype(jnp.bfloat16)
        o = jax.lax.dot_general(h.astype(jnp.bfloat16), w2,
                                (((1,), (0,)), ((), ())),
                                preferred_element_type=jnp.float32)
        o += b2_ref[k, 0]
        wcol = (jnp.where(i1_ref[...] == e, p1_ref[...], 0.0)
                + jnp.where(i2_ref[...] == e, p2_ref[...], 0.0))
        contrib += o * wcol
    acc_ref[...] = contrib

    @pl.when(s == N_STEPS - 1)
    def _write():
        out_ref[...] = contrib


@jax.jit
def kernel(x, gate_w, w1, b1, w2, b2):
    Bs, Ts, D = x.shape
    x_flat = x.reshape(-1, D)
    logits_t = pl.pallas_call(
        _logits_kernel,
        out_shape=jax.ShapeDtypeStruct((NUM_EXPERTS, N_TOKENS), jnp.float32),
    )(x_flat, gate_w)
    i1, i2, p1, p2 = _route(logits_t)
    out = pl.pallas_call(
        _moe_kernel,
        grid=(N_STEPS,),
        in_specs=[
            pl.BlockSpec((N_TOKENS, D_MODEL), lambda s: (0, 0)),
            pl.BlockSpec((E_BLK, D_MODEL, D_MODEL), lambda s: (s, 0, 0)),
            pl.BlockSpec((E_BLK, 1, D_MODEL), lambda s: (s, 0, 0)),
            pl.BlockSpec((E_BLK, D_MODEL, D_MODEL), lambda s: (s, 0, 0)),
            pl.BlockSpec((E_BLK, 1, D_MODEL), lambda s: (s, 0, 0)),
            pl.BlockSpec((N_TOKENS, 1), lambda s: (0, 0)),
            pl.BlockSpec((N_TOKENS, 1), lambda s: (0, 0)),
            pl.BlockSpec((N_TOKENS, 1), lambda s: (0, 0)),
            pl.BlockSpec((N_TOKENS, 1), lambda s: (0, 0)),
        ],
        out_specs=pl.BlockSpec((N_TOKENS, D_MODEL), lambda s: (0, 0)),
        out_shape=jax.ShapeDtypeStruct((N_TOKENS, D_MODEL), jnp.float32),
        scratch_shapes=[pltpu.VMEM((N_TOKENS, D_MODEL), jnp.float32)],
    )(x_flat, w1, b1[:, None, :], w2, b2[:, None, :],
      i1[:, None], i2[:, None], p1[:, None], p2[:, None])
    return out.reshape(Bs, Ts, D)


# speedup vs baseline: 1.2429x; 1.2429x over previous
import jax
import jax.numpy as jnp
from jax.experimental import pallas as pl
from jax.experimental.pallas import tpu as pltpu

D_MODEL = 768
NUM_EXPERTS = 64
N_TOKENS = 128
E_BLK = 2
N_STEPS = NUM_EXPERTS // E_BLK


def _moe_kernel(x_ref, gate_ref, w1_ref, b1_ref, w2_ref, b2_ref, out_ref,
                acc_ref, xb_ref, i1_ref, i2_ref, p1_ref, p2_ref):
    s = pl.program_id(0)

    @pl.when(s == 0)
    def _gating():
        x = x_ref[...]
        logits = jax.lax.dot_general(
            x, gate_ref[...], (((1,), (1,)), ((), ())),
            preferred_element_type=jnp.float32)
        eids = jax.lax.broadcasted_iota(jnp.int32, (N_TOKENS, NUM_EXPERTS), 1)
        big = jnp.int32(NUM_EXPERTS + 1)
        v1 = jnp.max(logits, axis=1, keepdims=True)
        i1 = jnp.min(jnp.where(logits == v1, eids, big), axis=1, keepdims=True)
        masked = jnp.where(eids == i1, -jnp.inf, logits)
        v2 = jnp.max(masked, axis=1, keepdims=True)
        i2 = jnp.min(jnp.where(masked == v2, eids, big), axis=1, keepdims=True)
        t = jnp.exp(v2 - v1)
        i1_ref[...] = i1
        i2_ref[...] = i2
        p1_ref[...] = 1.0 / (1.0 + t)
        p2_ref[...] = t / (1.0 + t)
        acc_ref[...] = jnp.zeros_like(acc_ref)
        xb_ref[...] = x.astype(jnp.bfloat16)

    xb = xb_ref[...]
    contrib = acc_ref[...]
    for k in range(E_BLK):
        e = s * E_BLK + k
        w1 = w1_ref[k].astype(jnp.bfloat16)
        h = jax.lax.dot_general(xb, w1, (((1,), (0,)), ((), ())),
                                preferred_element_type=jnp.float32)
        h += b1_ref[k, 0]
        h = h * 0.5 * (1.0 + jax.lax.erf(h * 0.7071067811865476))
        w2 = w2_ref[k].astype(jnp.bfloat16)
        o = jax.lax.dot_general(h.astype(jnp.bfloat16), w2,
                                (((1,), (0,)), ((), ())),
                                preferred_element_type=jnp.float32)
        o += b2_ref[k, 0]
        wcol = (jnp.where(i1_ref[...] == e, p1_ref[...], 0.0)
                + jnp.where(i2_ref[...] == e, p2_ref[...], 0.0))
        contrib += o * wcol
    acc_ref[...] = contrib

    @pl.when(s == N_STEPS - 1)
    def _write():
        out_ref[...] = contrib


@jax.jit
def kernel(x, gate_w, w1, b1, w2, b2):
    Bs, Ts, D = x.shape
    x_flat = x.reshape(-1, D)
    out = pl.pallas_call(
        _moe_kernel,
        grid=(N_STEPS,),
        in_specs=[
            pl.BlockSpec((N_TOKENS, D_MODEL), lambda s: (0, 0)),
            pl.BlockSpec((NUM_EXPERTS, D_MODEL), lambda s: (0, 0)),
            pl.BlockSpec((E_BLK, D_MODEL, D_MODEL), lambda s: (s, 0, 0)),
            pl.BlockSpec((E_BLK, 1, D_MODEL), lambda s: (s, 0, 0)),
            pl.BlockSpec((E_BLK, D_MODEL, D_MODEL), lambda s: (s, 0, 0)),
            pl.BlockSpec((E_BLK, 1, D_MODEL), lambda s: (s, 0, 0)),
        ],
        out_specs=pl.BlockSpec((N_TOKENS, D_MODEL), lambda s: (0, 0)),
        out_shape=jax.ShapeDtypeStruct((N_TOKENS, D_MODEL), jnp.float32),
        scratch_shapes=[pltpu.VMEM((N_TOKENS, D_MODEL), jnp.float32),
                        pltpu.VMEM((N_TOKENS, D_MODEL), jnp.bfloat16),
                        pltpu.VMEM((N_TOKENS, 1), jnp.int32),
                        pltpu.VMEM((N_TOKENS, 1), jnp.int32),
                        pltpu.VMEM((N_TOKENS, 1), jnp.float32),
                        pltpu.VMEM((N_TOKENS, 1), jnp.float32)],
    )(x_flat, gate_w, w1, b1[:, None, :], w2, b2[:, None, :])
    return out.reshape(Bs, Ts, D)


# speedup vs baseline: 1.2597x; 1.0136x over previous
import jax
import jax.numpy as jnp
from jax import lax
from jax.experimental import pallas as pl
from jax.experimental.pallas import tpu as pltpu

D_MODEL = 768
NUM_EXPERTS = 64
N_TOKENS = 128
E_BLK = 2
N_STEPS = NUM_EXPERTS // E_BLK
NBUF = 3


def _moe_kernel(x_ref, gate_ref, w1_hbm, b1_hbm, w2_hbm, b2_hbm, out_ref,
                w1_bufs, w2_bufs, b1_bufs, b2_bufs, acc_ref, sems):
    x = x_ref[...]
    logits = jax.lax.dot_general(
        x, gate_ref[...], (((1,), (1,)), ((), ())),
        preferred_element_type=jnp.float32)
    eids = jax.lax.broadcasted_iota(jnp.int32, (N_TOKENS, NUM_EXPERTS), 1)
    big = jnp.int32(NUM_EXPERTS + 1)
    v1 = jnp.max(logits, axis=1, keepdims=True)
    i1 = jnp.min(jnp.where(logits == v1, eids, big), axis=1, keepdims=True)
    masked = jnp.where(eids == i1, -jnp.inf, logits)
    v2 = jnp.max(masked, axis=1, keepdims=True)
    i2 = jnp.min(jnp.where(masked == v2, eids, big), axis=1, keepdims=True)
    t = jnp.exp(v2 - v1)
    p1 = 1.0 / (1.0 + t)
    p2 = t / (1.0 + t)

    xb = x.astype(jnp.bfloat16)

    def _start(s, b):
        sl = pl.ds(s * E_BLK, E_BLK)
        pltpu.make_async_copy(w1_hbm.at[sl], w1_bufs.at[b],
                              sems.at[b, 0]).start()
        pltpu.make_async_copy(w2_hbm.at[sl], w2_bufs.at[b],
                              sems.at[b, 1]).start()
        pltpu.make_async_copy(b1_hbm.at[sl], b1_bufs.at[b],
                              sems.at[b, 2]).start()
        pltpu.make_async_copy(b2_hbm.at[sl], b2_bufs.at[b],
                              sems.at[b, 3]).start()

    def _wait(s, b):
        sl = pl.ds(s * E_BLK, E_BLK)
        pltpu.make_async_copy(w1_hbm.at[sl], w1_bufs.at[b],
                              sems.at[b, 0]).wait()
        pltpu.make_async_copy(w2_hbm.at[sl], w2_bufs.at[b],
                              sems.at[b, 1]).wait()
        pltpu.make_async_copy(b1_hbm.at[sl], b1_bufs.at[b],
                              sems.at[b, 2]).wait()
        pltpu.make_async_copy(b2_hbm.at[sl], b2_bufs.at[b],
                              sems.at[b, 3]).wait()

    for b in range(NBUF):
        _start(b, b)

    acc_ref[...] = jnp.zeros_like(acc_ref)

    def _body(s, carry):
        b = s % NBUF
        _wait(s, b)
        contrib = acc_ref[...]
        for k in range(E_BLK):
            e = s * E_BLK + k
            w1 = w1_bufs[b, k].astype(jnp.bfloat16)
            h = jax.lax.dot_general(xb, w1, (((1,), (0,)), ((), ())),
                                    preferred_element_type=jnp.float32)
            h += b1_bufs[b, k]
            h = h * 0.5 * (1.0 + jax.lax.erf(h * 0.7071067811865476))
            w2 = w2_bufs[b, k].astype(jnp.bfloat16)
            o = jax.lax.dot_general(h.astype(jnp.bfloat16), w2,
                                    (((1,), (0,)), ((), ())),
                                    preferred_element_type=jnp.float32)
            o += b2_bufs[b, k]
            wcol = jnp.where(i1 == e, p1, 0.0) + jnp.where(i2 == e, p2, 0.0)
            contrib += o * wcol
        acc_ref[...] = contrib

        @pl.when(s + NBUF < N_STEPS)
        def _rearm():
            _start(s + NBUF, b)

        return carry

    lax.fori_loop(0, N_STEPS, _body, jnp.int32(0))
    out_ref[...] = acc_ref[...]


@jax.jit
def kernel(x, gate_w, w1, b1, w2, b2):
    Bs, Ts, D = x.shape
    x_flat = x.reshape(-1, D)
    out = pl.pallas_call(
        _moe_kernel,
        in_specs=[
            pl.BlockSpec(memory_space=pltpu.VMEM),
            pl.BlockSpec(memory_space=pltpu.VMEM),
            pl.BlockSpec(memory_space=pl.ANY),
            pl.BlockSpec(memory_space=pl.ANY),
            pl.BlockSpec(memory_space=pl.ANY),
            pl.BlockSpec(memory_space=pl.ANY),
        ],
        out_shape=jax.ShapeDtypeStruct((N_TOKENS, D_MODEL), jnp.float32),
        scratch_shapes=[
            pltpu.VMEM((NBUF, E_BLK, D_MODEL, D_MODEL), jnp.float32),
            pltpu.VMEM((NBUF, E_BLK, D_MODEL, D_MODEL), jnp.float32),
            pltpu.VMEM((NBUF, E_BLK, D_MODEL), jnp.float32),
            pltpu.VMEM((NBUF, E_BLK, D_MODEL), jnp.float32),
            pltpu.VMEM((N_TOKENS, D_MODEL), jnp.float32),
            pltpu.SemaphoreType.DMA((NBUF, 4)),
        ],
    )(x_flat, gate_w, w1, b1, w2, b2)
    return out.reshape(Bs, Ts, D)
